# baseline (speedup 1.0000x reference)
import functools

import jax
import jax.numpy as jnp
from jax import lax
from jax.experimental import pallas as pl
from jax.experimental.pallas import tpu as pltpu
from jax.experimental.pallas import tpu_sc as plsc

_NUM_CORES = 2
_NUM_SUBCORES = 16
_NUM_WORKERS = _NUM_CORES * _NUM_SUBCORES
_LANES = 16

_GROUP_L = 10


def _dequant_kernel(idx_hbm, w_hbm, s_hbm, out_hbm,
                    idx_v, w_v0, w_v1, s_v0, s_v1, out_v0, out_v1,
                    sem_g0, sem_g1, sem_o0, sem_o1):
  hist = out_hbm.shape[0]
  d8 = out_hbm.shape[1]
  bm = out_hbm.shape[4]
  rows_g = _GROUP_L * bm
  n_groups = hist // _GROUP_L

  w_bufs = (w_v0, w_v1)
  s_bufs = (s_v0, s_v1)
  g_sems = (sem_g0, sem_g1)
  o_bufs = (out_v0, out_v1)
  o_sems = (sem_o0, sem_o1)

  wid = lax.axis_index("s") * _NUM_CORES + lax.axis_index("c")

  iota = lax.iota(jnp.int32, _LANES)
  row_hi = iota >> 1
  row_lo = [(iota & 1) * 4 + j for j in range(4)]

  pltpu.sync_copy(idx_hbm.at[pl.ds(wid * hist * bm, hist * bm)], idx_v)

  def fire_gather(g):
    par = g % 2
    sl = idx_v.at[pl.ds(g * rows_g, rows_g)]
    pltpu.async_copy(w_hbm.at[sl], w_bufs[par], g_sems[par])
    pltpu.async_copy(s_hbm.at[sl], s_bufs[par].at[pl.ds(0, rows_g)],
                     g_sems[par])

  def wait_gather(g):
    par = g % 2
    sl = idx_v.at[pl.ds(g * rows_g, rows_g)]
    pltpu.make_async_copy(w_hbm.at[sl], w_bufs[par], g_sems[par]).wait()
    pltpu.make_async_copy(s_hbm.at[sl], s_bufs[par].at[pl.ds(0, rows_g)],
                          g_sems[par]).wait()

  fire_gather(0)
  for g in range(n_groups):
    if g + 1 < n_groups:
      fire_gather(g + 1)
    wait_gather(g)
    w_v, s_v = w_bufs[g % 2], s_bufs[g % 2]
    for li in range(_GROUP_L):
      l = g * _GROUP_L + li
      out_v, sem_o = o_bufs[li % 2], o_sems[li % 2]
      if l >= 2:
        pltpu.make_async_copy(out_v, out_hbm.at[0, :, wid], sem_o).wait()

      def row_body(r, _):
        packed = w_v[li * bm + r, 0]
        words = plsc.bitcast(packed, jnp.int32)
        s_vec = s_v[pl.ds(li * bm + r, _LANES)]
        s = jnp.broadcast_to(s_vec[0], (_LANES,))
        r_splat = jnp.full((_LANES,), r, jnp.int32)
        for j in range(4):
          v = (words << (24 - 8 * j)) >> 24 if j < 3 else words >> 24
          plsc.store_scatter(out_v, [row_hi, row_lo[j], r_splat],
                             v.astype(jnp.float32) * s)
        return 0

      lax.fori_loop(0, bm, row_body, 0)
      pltpu.async_copy(out_v, out_hbm.at[l, :, wid], sem_o)

  for par in range(2):
    pltpu.make_async_copy(o_bufs[par], out_hbm.at[0, :, wid],
                          o_sems[par]).wait()


def kernel(input, weight_int8, scale):
  batch, hist = input.shape
  vocab, dim = weight_int8.shape
  bm = batch // _NUM_WORKERS
  rows_g = _GROUP_L * bm

  idx_r = (input.T.astype(jnp.int32)
           .reshape(hist, _NUM_WORKERS, bm)
           .transpose(1, 0, 2)
           .reshape(batch * hist))
  scale_flat = scale.reshape(vocab)
  w_packed = weight_int8.reshape(vocab, 1, dim)

  mesh = plsc.VectorSubcoreMesh(core_axis_name="c", subcore_axis_name="s")
  run = pl.kernel(
      _dequant_kernel,
      out_type=jax.ShapeDtypeStruct(
          (hist, dim // 8, _NUM_WORKERS, 8, bm), jnp.float32),
      mesh=mesh,
      compiler_params=pltpu.CompilerParams(
          needs_layout_passes=False, use_tc_tiling_on_sc=False),
      scratch_types=[
          pltpu.VMEM((hist * bm,), jnp.int32),
          pltpu.VMEM((rows_g, 1, dim), jnp.int8),
          pltpu.VMEM((rows_g, 1, dim), jnp.int8),
          pltpu.VMEM((rows_g + _LANES,), jnp.float32),
          pltpu.VMEM((rows_g + _LANES,), jnp.float32),
          pltpu.VMEM((dim // 8, 8, bm), jnp.float32),
          pltpu.VMEM((dim // 8, 8, bm), jnp.float32),
          pltpu.SemaphoreType.DMA,
          pltpu.SemaphoreType.DMA,
          pltpu.SemaphoreType.DMA,
          pltpu.SemaphoreType.DMA,
      ],
  )
  out5 = run(idx_r, w_packed, scale_flat)
  return out5.transpose(2, 4, 0, 1, 3).reshape(batch, hist, dim)


# speedup vs baseline: 1.8024x; 1.8024x over previous
import functools

import jax
import jax.numpy as jnp
from jax import lax
from jax.experimental import pallas as pl
from jax.experimental.pallas import tpu as pltpu
from jax.experimental.pallas import tpu_sc as plsc

_NUM_CORES = 2
_NUM_SUBCORES = 16
_NUM_WORKERS = _NUM_CORES * _NUM_SUBCORES
_LANES = 16

_GROUP_L = 10


def _dequant_kernel(idx_hbm, w_hbm, s_hbm, out_hbm,
                    idx_v, w_v0, w_v1, s_v0, s_v1, out_v0, out_v1,
                    sem_g0, sem_g1, sem_o0, sem_o1):
  hist = out_hbm.shape[0]
  d8 = out_hbm.shape[1]
  bm = out_hbm.shape[4]
  rows_g = _GROUP_L * bm
  n_groups = hist // _GROUP_L

  w_bufs = (w_v0, w_v1)
  s_bufs = (s_v0, s_v1)
  g_sems = (sem_g0, sem_g1)
  o_bufs = (out_v0, out_v1)
  o_sems = (sem_o0, sem_o1)

  wid = lax.axis_index("s") * _NUM_CORES + lax.axis_index("c")

  iota = lax.iota(jnp.int32, _LANES)
  w_idx = [q * 4 + (iota >> 2) for q in range(4)]
  lsh = 24 - 8 * (iota & 3)
  d8_idx = [2 * q + (iota >> 3) for q in range(4)]
  dm_idx = iota & 7

  pltpu.sync_copy(idx_hbm.at[pl.ds(wid * hist * bm, hist * bm)], idx_v)

  def fire_gather(g):
    par = g % 2
    sl = idx_v.at[pl.ds(g * rows_g, rows_g)]
    pltpu.async_copy(w_hbm.at[sl], w_bufs[par], g_sems[par])
    pltpu.async_copy(s_hbm.at[sl], s_bufs[par].at[pl.ds(0, rows_g)],
                     g_sems[par])

  def wait_gather(g):
    par = g % 2
    sl = idx_v.at[pl.ds(g * rows_g, rows_g)]
    pltpu.make_async_copy(w_hbm.at[sl], w_bufs[par], g_sems[par]).wait()
    pltpu.make_async_copy(s_hbm.at[sl], s_bufs[par].at[pl.ds(0, rows_g)],
                          g_sems[par]).wait()

  fire_gather(0)
  for g in range(n_groups):
    if g + 1 < n_groups:
      fire_gather(g + 1)
    wait_gather(g)
    w_v, s_v = w_bufs[g % 2], s_bufs[g % 2]
    for li in range(_GROUP_L):
      l = g * _GROUP_L + li
      out_v, sem_o = o_bufs[li % 2], o_sems[li % 2]
      if l >= 2:
        pltpu.make_async_copy(out_v.at[:, :, pl.ds(0, bm)],
                              out_hbm.at[0, :, wid], sem_o).wait()

      def row_body(r, _):
        packed = w_v[li * bm + r, 0]
        words = plsc.bitcast(packed, jnp.int32)
        s_vec = s_v[pl.ds(li * bm + r, _LANES)]
        s = jnp.broadcast_to(s_vec[0], (_LANES,))
        r_splat = jnp.full((_LANES,), r, jnp.int32)
        for q in range(4):
          wq = words[w_idx[q]]
          v = (wq << lsh) >> 24
          plsc.store_scatter(out_v, [d8_idx[q], dm_idx, r_splat],
                             v.astype(jnp.float32) * s)
        return 0

      lax.fori_loop(0, bm, row_body, 0)
      pltpu.async_copy(out_v.at[:, :, pl.ds(0, bm)],
                       out_hbm.at[l, :, wid], sem_o)

  for par in range(2):
    pltpu.make_async_copy(o_bufs[par].at[:, :, pl.ds(0, bm)],
                          out_hbm.at[0, :, wid], o_sems[par]).wait()


def kernel(input, weight_int8, scale):
  batch, hist = input.shape
  vocab, dim = weight_int8.shape
  bm = batch // _NUM_WORKERS
  rows_g = _GROUP_L * bm

  idx_r = (input.T.astype(jnp.int32)
           .reshape(hist, _NUM_WORKERS, bm)
           .transpose(1, 0, 2)
           .reshape(batch * hist))
  scale_flat = scale.reshape(vocab)
  w_packed = weight_int8.reshape(vocab, 1, dim)

  mesh = plsc.VectorSubcoreMesh(core_axis_name="c", subcore_axis_name="s")
  run = pl.kernel(
      _dequant_kernel,
      out_type=jax.ShapeDtypeStruct(
          (hist, dim // 8, _NUM_WORKERS, 8, bm), jnp.float32),
      mesh=mesh,
      compiler_params=pltpu.CompilerParams(
          needs_layout_passes=False, use_tc_tiling_on_sc=False),
      scratch_types=[
          pltpu.VMEM((hist * bm,), jnp.int32),
          pltpu.VMEM((rows_g, 1, dim), jnp.int8),
          pltpu.VMEM((rows_g, 1, dim), jnp.int8),
          pltpu.VMEM((rows_g + _LANES,), jnp.float32),
          pltpu.VMEM((rows_g + _LANES,), jnp.float32),
          pltpu.VMEM((dim // 8, 8, bm + 1), jnp.float32),
          pltpu.VMEM((dim // 8, 8, bm + 1), jnp.float32),
          pltpu.SemaphoreType.DMA,
          pltpu.SemaphoreType.DMA,
          pltpu.SemaphoreType.DMA,
          pltpu.SemaphoreType.DMA,
      ],
  )
  out5 = run(idx_r, w_packed, scale_flat)
  return out5.transpose(2, 4, 0, 1, 3).reshape(batch, hist, dim)


# speedup vs baseline: 1.8187x; 1.0090x over previous
import functools

import jax
import jax.numpy as jnp
from jax import lax
from jax.experimental import pallas as pl
from jax.experimental.pallas import tpu as pltpu
from jax.experimental.pallas import tpu_sc as plsc

_NUM_CORES = 2
_NUM_SUBCORES = 16
_NUM_WORKERS = _NUM_CORES * _NUM_SUBCORES
_LANES = 16

_GROUP_L = 10
_UNROLL = 4


def _dequant_kernel(idx_hbm, w_hbm, s_hbm, out_hbm,
                    idx_v, w_v0, w_v1, s_v0, s_v1, out_v0, out_v1,
                    sem_g0, sem_g1, sem_o0, sem_o1):
  hist = out_hbm.shape[0]
  bm = out_hbm.shape[4]
  rows_g = _GROUP_L * bm
  n_groups = hist // _GROUP_L

  w_bufs = (w_v0, w_v1)
  s_bufs = (s_v0, s_v1)
  g_sems = (sem_g0, sem_g1)
  o_bufs = (out_v0, out_v1)
  o_sems = (sem_o0, sem_o1)

  wid = lax.axis_index("s") * _NUM_CORES + lax.axis_index("c")

  iota = lax.iota(jnp.int32, _LANES)
  w_idx = [q * 4 + (iota >> 2) for q in range(4)]
  lsh = 24 - 8 * (iota & 3)
  d8_idx = [2 * q + (iota >> 3) for q in range(4)]
  dm_idx = iota & 7

  pltpu.sync_copy(idx_hbm.at[pl.ds(wid * hist * bm, hist * bm)], idx_v)

  def fire_gather(g):
    par = g % 2
    sl = idx_v.at[pl.ds(g * rows_g, rows_g)]
    pltpu.async_copy(w_hbm.at[sl], w_bufs[par], g_sems[par])
    pltpu.async_copy(s_hbm.at[sl], s_bufs[par].at[pl.ds(0, rows_g)],
                     g_sems[par])

  def wait_gather(g):
    par = g % 2
    sl = idx_v.at[pl.ds(g * rows_g, rows_g)]
    pltpu.make_async_copy(w_hbm.at[sl], w_bufs[par], g_sems[par]).wait()
    pltpu.make_async_copy(s_hbm.at[sl], s_bufs[par].at[pl.ds(0, rows_g)],
                          g_sems[par]).wait()

  def drain_out(par):
    pltpu.make_async_copy(o_bufs[par].at[:, :, pl.ds(0, bm)],
                          out_hbm.at[0, :, wid], o_sems[par]).wait()

  fire_gather(0)
  for g in range(n_groups):
    if g + 1 < n_groups:
      fire_gather(g + 1)
    wait_gather(g)
    w_v, s_v = w_bufs[g % 2], s_bufs[g % 2]

    def pair_body(t2, _):
      for par in range(2):
        l_loc = 2 * t2 + par
        l = g * _GROUP_L + l_loc
        out_v, sem_o = o_bufs[par], o_sems[par]

        if g > 0:
          drain_out(par)
        else:
          @pl.when(t2 > 0)
          def _():
            drain_out(par)

        def row_body(r4, _):
          b0 = r4 * _UNROLL
          row0 = l_loc * bm + b0
          s_vec = s_v[pl.ds(row0, _LANES)]
          for k in range(_UNROLL):
            packed = w_v[row0 + k, 0]
            words = plsc.bitcast(packed, jnp.int32)
            s = jnp.broadcast_to(s_vec[k], (_LANES,))
            r_splat = jnp.full((_LANES,), b0 + k, jnp.int32)
            for q in range(4):
              wq = words[w_idx[q]]
              v = (wq << lsh) >> 24
              plsc.store_scatter(out_v, [d8_idx[q], dm_idx, r_splat],
                                 v.astype(jnp.float32) * s)
          return 0

        lax.fori_loop(0, bm // _UNROLL, row_body, 0)
        pltpu.async_copy(out_v.at[:, :, pl.ds(0, bm)],
                         out_hbm.at[l, :, wid], sem_o)
      return 0

    lax.fori_loop(0, _GROUP_L // 2, pair_body, 0)

  for par in range(2):
    drain_out(par)


def kernel(input, weight_int8, scale):
  batch, hist = input.shape
  vocab, dim = weight_int8.shape
  bm = batch // _NUM_WORKERS
  rows_g = _GROUP_L * bm

  idx_r = (input.T.astype(jnp.int32)
           .reshape(hist, _NUM_WORKERS, bm)
           .transpose(1, 0, 2)
           .reshape(batch * hist))
  scale_flat = scale.reshape(vocab)
  w_packed = weight_int8.reshape(vocab, 1, dim)

  mesh = plsc.VectorSubcoreMesh(core_axis_name="c", subcore_axis_name="s")
  run = pl.kernel(
      _dequant_kernel,
      out_type=jax.ShapeDtypeStruct(
          (hist, dim // 8, _NUM_WORKERS, 8, bm), jnp.float32),
      mesh=mesh,
      compiler_params=pltpu.CompilerParams(
          needs_layout_passes=False, use_tc_tiling_on_sc=False),
      scratch_types=[
          pltpu.VMEM((hist * bm,), jnp.int32),
          pltpu.VMEM((rows_g, 1, dim), jnp.int8),
          pltpu.VMEM((rows_g, 1, dim), jnp.int8),
          pltpu.VMEM((rows_g + _LANES,), jnp.float32),
          pltpu.VMEM((rows_g + _LANES,), jnp.float32),
          pltpu.VMEM((dim // 8, 8, bm + 1), jnp.float32),
          pltpu.VMEM((dim // 8, 8, bm + 1), jnp.float32),
          pltpu.SemaphoreType.DMA,
          pltpu.SemaphoreType.DMA,
          pltpu.SemaphoreType.DMA,
          pltpu.SemaphoreType.DMA,
      ],
  )
  out5 = run(idx_r, w_packed, scale_flat)
  return out5.transpose(2, 4, 0, 1, 3).reshape(batch, hist, dim)
